# baseline (speedup 1.0000x reference)
import jax
import jax.numpy as jnp
from jax.experimental import pallas as pl

_TB = 64


def _fused_ae_kernel(x_ref, gv_ref, g_ref, we_ref, be_ref, wd_ref, bd_ref, out_ref):
    xa = x_ref[...] + g_ref[...] * gv_ref[...]
    h = jnp.dot(xa, we_ref[...], preferred_element_type=jnp.float32)
    h = jnp.maximum(h + be_ref[...], 0.0)
    out = jnp.dot(h, wd_ref[...], preferred_element_type=jnp.float32)
    out_ref[...] = out + bd_ref[...]


def kernel(x, genre_vec, g, W_enc, b_enc, W_dec, b_dec):
    B, SIZE = x.shape
    HIDDEN = W_enc.shape[1]
    b_enc2 = b_enc.reshape(1, HIDDEN)
    b_dec2 = b_dec.reshape(1, SIZE)

    grid = (B // _TB,)
    return pl.pallas_call(
        _fused_ae_kernel,
        grid=grid,
        in_specs=[
            pl.BlockSpec((_TB, SIZE), lambda i: (i, 0)),
            pl.BlockSpec((_TB, SIZE), lambda i: (i, 0)),
            pl.BlockSpec((1, SIZE), lambda i: (0, 0)),
            pl.BlockSpec((SIZE, HIDDEN), lambda i: (0, 0)),
            pl.BlockSpec((1, HIDDEN), lambda i: (0, 0)),
            pl.BlockSpec((HIDDEN, SIZE), lambda i: (0, 0)),
            pl.BlockSpec((1, SIZE), lambda i: (0, 0)),
        ],
        out_specs=pl.BlockSpec((_TB, SIZE), lambda i: (i, 0)),
        out_shape=jax.ShapeDtypeStruct((B, SIZE), jnp.float32),
    )(x, genre_vec, g, W_enc, b_enc2, W_dec, b_dec2)


# speedup vs baseline: 1.0940x; 1.0940x over previous
import jax
import jax.numpy as jnp
from jax.experimental import pallas as pl
from jax.experimental.pallas import tpu as pltpu

_TB = 128


def _fused_ae_kernel(x_ref, gv_ref, g_ref, we_ref, be_ref, wd_ref, bd_ref, out_ref):
    xa = x_ref[...] + g_ref[...] * gv_ref[...]
    h = jnp.dot(xa, we_ref[...], preferred_element_type=jnp.float32)
    h = jnp.maximum(h + be_ref[...], 0.0)
    out = jnp.dot(h, wd_ref[...], preferred_element_type=jnp.float32)
    out_ref[...] = out + bd_ref[...]


def kernel(x, genre_vec, g, W_enc, b_enc, W_dec, b_dec):
    B, SIZE = x.shape
    HIDDEN = W_enc.shape[1]
    b_enc2 = b_enc.reshape(1, HIDDEN)
    b_dec2 = b_dec.reshape(1, SIZE)

    grid = (B // _TB,)
    return pl.pallas_call(
        _fused_ae_kernel,
        grid=grid,
        in_specs=[
            pl.BlockSpec((_TB, SIZE), lambda i: (i, 0)),
            pl.BlockSpec((_TB, SIZE), lambda i: (i, 0)),
            pl.BlockSpec((1, SIZE), lambda i: (0, 0)),
            pl.BlockSpec((SIZE, HIDDEN), lambda i: (0, 0)),
            pl.BlockSpec((1, HIDDEN), lambda i: (0, 0)),
            pl.BlockSpec((HIDDEN, SIZE), lambda i: (0, 0)),
            pl.BlockSpec((1, SIZE), lambda i: (0, 0)),
        ],
        out_specs=pl.BlockSpec((_TB, SIZE), lambda i: (i, 0)),
        out_shape=jax.ShapeDtypeStruct((B, SIZE), jnp.float32),
        compiler_params=pltpu.CompilerParams(
            dimension_semantics=("parallel",),
        ),
    )(x, genre_vec, g, W_enc, b_enc2, W_dec, b_dec2)
